# baseline (speedup 1.0000x reference)
import functools

import jax
import jax.numpy as jnp
from jax import lax
from jax.experimental import pallas as pl
from jax.experimental.pallas import tpu as pltpu
from jax.experimental.pallas import tpu_sc as plsc

_NUM_CORES = 2
_NUM_SUBCORES = 16
_NUM_WORKERS = _NUM_CORES * _NUM_SUBCORES


def kernel(x, emb):
    seq_len = x.shape[1]
    emb_dim = emb.shape[1]
    rows_per_worker = seq_len // _NUM_WORKERS

    mesh = plsc.VectorSubcoreMesh(core_axis_name="c", subcore_axis_name="s")

    @functools.partial(
        pl.kernel,
        mesh=mesh,
        out_type=jax.ShapeDtypeStruct((seq_len, emb_dim), emb.dtype),
    )
    def sc_copy(emb_hbm, out_hbm):
        wid = lax.axis_index("s") * _NUM_CORES + lax.axis_index("c")
        base = wid * rows_per_worker
        pltpu.sync_copy(
            emb_hbm.at[pl.ds(base, rows_per_worker)],
            out_hbm.at[pl.ds(base, rows_per_worker)],
        )

    return sc_copy(emb)[None]


# speedup vs baseline: 16.3334x; 16.3334x over previous
import functools

import jax
import jax.numpy as jnp
from jax import lax
from jax.experimental import pallas as pl
from jax.experimental.pallas import tpu as pltpu
from jax.experimental.pallas import tpu_sc as plsc

_NUM_CORES = 2
_NUM_SUBCORES = 16
_NUM_WORKERS = _NUM_CORES * _NUM_SUBCORES


def kernel(x, emb):
    seq_len = x.shape[1]
    emb_dim = emb.shape[1]
    rows_per_worker = seq_len // _NUM_WORKERS

    mesh = plsc.VectorSubcoreMesh(core_axis_name="c", subcore_axis_name="s")

    chunk = 32
    n_chunks = rows_per_worker // chunk

    @functools.partial(
        pl.kernel,
        mesh=mesh,
        out_type=jax.ShapeDtypeStruct((seq_len, emb_dim), emb.dtype),
        scratch_types=[
            pltpu.VMEM((2, chunk, emb_dim), jnp.float32),
            pltpu.SemaphoreType.DMA,
            pltpu.SemaphoreType.DMA,
        ],
    )
    def sc_copy(emb_hbm, out_hbm, buf, sem_in, sem_out):
        wid = lax.axis_index("s") * _NUM_CORES + lax.axis_index("c")
        base = wid * rows_per_worker

        def in_cp(i, b):
            return pltpu.make_async_copy(
                emb_hbm.at[pl.ds(base + i * chunk, chunk)], buf.at[b], sem_in
            )

        def out_cp(i, b):
            return pltpu.make_async_copy(
                buf.at[b], out_hbm.at[pl.ds(base + i * chunk, chunk)], sem_out
            )

        in_cp(0, 0).start()
        for i in range(n_chunks):
            cur = i % 2
            in_cp(i, cur).wait()
            out_cp(i, cur).start()
            if i + 1 < n_chunks:
                if i >= 1:
                    out_cp(i - 1, 1 - cur).wait()
                in_cp(i + 1, 1 - cur).start()
        out_cp(n_chunks - 1, (n_chunks - 1) % 2).wait()

    return sc_copy(emb)[None]


# speedup vs baseline: 17.1466x; 1.0498x over previous
import functools

import jax
import jax.numpy as jnp
from jax import lax
from jax.experimental import pallas as pl
from jax.experimental.pallas import tpu as pltpu
from jax.experimental.pallas import tpu_sc as plsc

_NUM_CORES = 2
_NUM_SUBCORES = 16
_NUM_WORKERS = _NUM_CORES * _NUM_SUBCORES


def kernel(x, emb):
    seq_len = x.shape[1]
    emb_dim = emb.shape[1]
    rows_per_worker = seq_len // _NUM_WORKERS

    mesh = plsc.VectorSubcoreMesh(core_axis_name="c", subcore_axis_name="s")

    chunk = 16
    n_chunks = rows_per_worker // chunk
    nbuf = 7

    @functools.partial(
        pl.kernel,
        mesh=mesh,
        out_type=jax.ShapeDtypeStruct((seq_len, emb_dim), emb.dtype),
        scratch_types=[
            pltpu.VMEM((nbuf, chunk, emb_dim), jnp.float32),
            pltpu.SemaphoreType.DMA,
            pltpu.SemaphoreType.DMA,
        ],
    )
    def sc_copy(emb_hbm, out_hbm, buf, sem_in, sem_out):
        wid = lax.axis_index("s") * _NUM_CORES + lax.axis_index("c")
        base = wid * rows_per_worker

        def in_cp(i, b):
            return pltpu.make_async_copy(
                emb_hbm.at[pl.ds(base + i * chunk, chunk)], buf.at[b], sem_in
            )

        def out_cp(i, b):
            return pltpu.make_async_copy(
                buf.at[b], out_hbm.at[pl.ds(base + i * chunk, chunk)], sem_out
            )

        for j in range(min(nbuf, n_chunks)):
            in_cp(j, j).start()
        for i in range(n_chunks):
            b = i % nbuf
            in_cp(i, b).wait()
            out_cp(i, b).start()
            j = i + nbuf
            if j < n_chunks:
                out_cp(j - nbuf, b).wait()
                in_cp(j, b).start()
        for i in range(max(0, n_chunks - nbuf), n_chunks):
            out_cp(i, i % nbuf).wait()

    return sc_copy(emb)[None]


# speedup vs baseline: 17.2040x; 1.0033x over previous
import functools

import jax
import jax.numpy as jnp
from jax import lax
from jax.experimental import pallas as pl
from jax.experimental.pallas import tpu as pltpu
from jax.experimental.pallas import tpu_sc as plsc

_NUM_CORES = 2
_NUM_SUBCORES = 16
_NUM_WORKERS = _NUM_CORES * _NUM_SUBCORES


def kernel(x, emb):
    seq_len = x.shape[1]
    emb_dim = emb.shape[1]
    rows_per_worker = seq_len // _NUM_WORKERS

    mesh = plsc.VectorSubcoreMesh(core_axis_name="c", subcore_axis_name="s")

    chunk = 32
    n_chunks = rows_per_worker // chunk
    nbuf = 3

    @functools.partial(
        pl.kernel,
        mesh=mesh,
        out_type=jax.ShapeDtypeStruct((seq_len, emb_dim), emb.dtype),
        scratch_types=[
            pltpu.VMEM((nbuf, chunk, emb_dim), jnp.float32),
            pltpu.SemaphoreType.DMA,
            pltpu.SemaphoreType.DMA,
        ],
    )
    def sc_copy(emb_hbm, out_hbm, buf, sem_in, sem_out):
        wid = lax.axis_index("s") * _NUM_CORES + lax.axis_index("c")
        base = wid * rows_per_worker

        def in_cp(i, b):
            return pltpu.make_async_copy(
                emb_hbm.at[pl.ds(base + i * chunk, chunk)], buf.at[b], sem_in
            )

        def out_cp(i, b):
            return pltpu.make_async_copy(
                buf.at[b], out_hbm.at[pl.ds(base + i * chunk, chunk)], sem_out
            )

        for j in range(min(nbuf, n_chunks)):
            in_cp(j, j).start()
        for i in range(n_chunks):
            b = i % nbuf
            in_cp(i, b).wait()
            out_cp(i, b).start()
            j = i + nbuf
            if j < n_chunks:
                out_cp(j - nbuf, b).wait()
                in_cp(j, b).start()
        for i in range(max(0, n_chunks - nbuf), n_chunks):
            out_cp(i, i % nbuf).wait()

    return sc_copy(emb)[None]
